# baseline (speedup 1.0000x reference)
import jax
import jax.numpy as jnp
from jax.experimental import pallas as pl

_RELS = [('variable','value'),('variable','operator'),('variable','constraint'),('operator','constraint'),('constraint','constraint'),('value','variable'),('operator','variable'),('constraint','variable'),('constraint','operator')]
_B = 64
_L = 2


def _gat(x_src, x_dst, src, dst, Ws, Wd, a_s, a_d, b):
    hs = x_src @ Ws
    hd = x_dst @ Wd
    e = (hs @ a_s)[src] + (hd @ a_d)[dst]
    e = jnp.where(e > 0, e, 0.2 * e)
    nd = x_dst.shape[0]
    m = jax.ops.segment_max(e, dst, num_segments=nd)
    m = jnp.where(jnp.isfinite(m), m, 0.0)
    ee = jnp.exp(e - m[dst])
    den = jax.ops.segment_sum(ee, dst, num_segments=nd)
    alpha = ee / (den[dst] + 1e-16)
    return jax.ops.segment_sum(alpha[:, None] * hs[src], dst, num_segments=nd) + b


def kernel(x_variable, x_value, x_operator, x_constraint, edge_index_0, edge_index_1, edge_index_2, edge_index_3, edge_index_4, edge_index_5, edge_index_6, edge_index_7, edge_index_8, batch_variable, batch_constraint, Wsrc, Wdst, att_src, att_dst, bias, linW, linb):
    xs = {'variable': x_variable, 'value': x_value, 'operator': x_operator, 'constraint': x_constraint}
    edges = [edge_index_0, edge_index_1, edge_index_2, edge_index_3, edge_index_4, edge_index_5, edge_index_6, edge_index_7, edge_index_8]
    x = dict(xs)
    for l in range(_L):
        acc = {}
        for r, (s, d) in enumerate(_RELS):
            o = _gat(x[s], x[d], edges[r][0], edges[r][1], Wsrc[l, r], Wdst[l, r], att_src[l, r], att_dst[l, r], bias[l, r])
            acc[d] = o if d not in acc else acc[d] + o
        x = {t: jax.nn.relu(v) for t, v in acc.items()}
    def pool(f, bt):
        s = jax.ops.segment_sum(f, bt, num_segments=_B)
        c = jax.ops.segment_sum(jnp.ones((f.shape[0],), f.dtype), bt, num_segments=_B)
        return s / jnp.maximum(c, 1.0)[:, None]
    vp = pool(x['variable'], batch_variable)
    cp = pool(x['constraint'], batch_constraint)
    return jnp.concatenate([vp, cp], 1) @ linW + linb


# speedup vs baseline: 2.5974x; 2.5974x over previous
import functools
import jax
import jax.numpy as jnp
from jax import lax
from jax.experimental import pallas as pl
from jax.experimental.pallas import tpu as pltpu
from jax.experimental.pallas import tpu_sc as plsc

_RELS = [('variable', 'value'), ('variable', 'operator'), ('variable', 'constraint'),
         ('operator', 'constraint'), ('constraint', 'constraint'), ('value', 'variable'),
         ('operator', 'variable'), ('constraint', 'variable'), ('constraint', 'operator')]
_NT = {'variable': 50000, 'value': 2000, 'operator': 500, 'constraint': 10000}
_TYPES = ['variable', 'value', 'operator', 'constraint']
_B = 64
_L = 2
_E = 64000
_H = 128
_C = 128
_EPAD = _E + _C

_GROUPS = {'value': [0], 'operator': [1, 8], 'constraint': [2, 3, 4], 'variable': [5, 6, 7]}
_WCFG = {'variable': (128, 512), 'constraint': (32, 384), 'value': (32, 128), 'operator': (32, 128)}


def _round_up(n, m):
    return (n + m - 1) // m * m


def _mm_body(x_ref, w_ref, o_ref):
    o_ref[...] = jnp.dot(x_ref[...], w_ref[...], preferred_element_type=jnp.float32)


def _matmul(x, w):
    npad, cp = x.shape[0], w.shape[1]
    nb = npad // 512
    return pl.pallas_call(
        _mm_body,
        grid=(nb,),
        in_specs=[pl.BlockSpec((512, _H), lambda i: (i, 0)),
                  pl.BlockSpec((_H, cp), lambda i: (0, 0))],
        out_specs=pl.BlockSpec((512, cp), lambda i: (i, 0)),
        out_shape=jax.ShapeDtypeStruct((npad, cp), jnp.float32),
    )(x, w)


def _pool_body(ids_ref, x_ref, o_ref):
    @pl.when(pl.program_id(0) == 0)
    def _():
        o_ref[...] = jnp.zeros_like(o_ref)
    ids = ids_ref[0, 0, :]
    mask = (ids[None, :] == lax.broadcasted_iota(jnp.int32, (_B, 512), 0)).astype(jnp.float32)
    o_ref[...] += jnp.dot(mask, x_ref[...], preferred_element_type=jnp.float32)


def _pool_sums(x, ids):
    n = x.shape[0]
    npad = _round_up(n, 512)
    xp = jnp.concatenate([x, jnp.ones((n, 1), jnp.float32)], axis=1)
    xp = jnp.pad(xp, ((0, npad - n), (0, 256 - _H - 1)))
    idp = jnp.pad(ids.astype(jnp.int32), (0, npad - n), constant_values=_B)
    idp = idp.reshape(npad // 512, 1, 512)
    nb = npad // 512
    return pl.pallas_call(
        _pool_body,
        grid=(nb,),
        in_specs=[pl.BlockSpec((1, 1, 512), lambda i: (i, 0, 0)),
                  pl.BlockSpec((512, 256), lambda i: (i, 0))],
        out_specs=pl.BlockSpec((_B, 256), lambda i: (0, 0)),
        out_shape=jax.ShapeDtypeStruct((_B, 256), jnp.float32),
    )(idp, xp)


def _head_body(sv_ref, sc_ref, w_ref, b_ref, o_ref):
    sv = sv_ref[...]
    sc = sc_ref[...]
    cv = jnp.maximum(sv[:, _H:_H + 1], 1.0)
    cc = jnp.maximum(sc[:, _H:_H + 1], 1.0)
    pooled = jnp.concatenate([sv[:, :_H] / cv, sc[:, :_H] / cc], axis=1)
    o_ref[...] = jnp.dot(pooled, w_ref[...], preferred_element_type=jnp.float32) + b_ref[...]


def _head(sv, sc, lin_w, lin_b):
    wp = jnp.pad(lin_w, ((0, 0), (0, _H - lin_w.shape[1])))
    bp = jnp.pad(lin_b, (0, _H - lin_b.shape[0]))[None, :]
    out = pl.pallas_call(
        _head_body,
        out_shape=jax.ShapeDtypeStruct((_B, _H), jnp.float32),
    )(sv, sc, wp, bp)
    return out[:, :lin_w.shape[1]]


def _make_sc_kernel(k_rels, w_workers, rows, ns_cat, op_cols):
    mesh = plsc.VectorSubcoreMesh(core_axis_name="c", subcore_axis_name="s")
    nrounds = w_workers // 32
    ndp = w_workers * rows

    @functools.partial(
        pl.kernel,
        mesh=mesh,
        compiler_params=pltpu.CompilerParams(needs_layout_passes=False),
        out_type=jax.ShapeDtypeStruct((ndp * _H,), jnp.float32),
        scratch_types=[
            pltpu.VMEM((rows * _H,), jnp.float32),
            pltpu.VMEM((_C, _H), jnp.float32),
            pltpu.VMEM((_C,), jnp.int32),
            pltpu.VMEM((_C,), jnp.int32),
            pltpu.VMEM((_C,), jnp.float32),
            pltpu.VMEM((_C,), jnp.float32),
            pltpu.VMEM((_C,), jnp.int32),
            pltpu.VMEM((rows,), jnp.float32),
            pltpu.VMEM((rows,), jnp.float32),
            pltpu.VMEM((len(k_rels), op_cols), jnp.int32),
            pltpu.VMEM((16,), jnp.float32),
            pltpu.SemaphoreType.DMA,
        ],
    )
    def sc_kernel(hs_hbm, ssrc_hbm, sdst_hbm, src_hbm, dst_hbm, offs_hbm, m_hbm,
                  out_hbm,
                  out_v, rows_v, srcidx_v, dstidx_v, ssrcg_v, ee_v, dl_v,
                  sdst_v, den_v, offs_s, m_s, sem):
        wid = lax.axis_index("s") * 2 + lax.axis_index("c")
        pltpu.sync_copy(offs_hbm, offs_s)
        pltpu.sync_copy(m_hbm, m_s)
        lanes = lax.iota(jnp.int32, 16)
        lane0 = lanes == 0
        zero16 = jnp.zeros((16,), jnp.float32)
        mvec = m_s[...]

        def edge_scalars(kk, base, d0, e0, e1, j):
            gidx = base + j * 16 + lanes
            valid = (gidx >= e0) & (gidx < e1)
            dst16 = dstidx_v[pl.ds(j * 16, 16)]
            dl16 = jnp.clip(dst16 - d0, 0, rows - 1)
            ssrc16 = ssrcg_v[pl.ds(j * 16, 16)]
            sdst16 = plsc.load_gather(sdst_v, [dl16])
            s = ssrc16 + sdst16
            e = jnp.where(s > 0, s, 0.2 * s)
            ee = jnp.exp(e - mvec[kk])
            ee = jnp.where(valid, ee, 0.0)
            return dl16, ee

        def load_chunk(kk, base):
            pltpu.sync_copy(src_hbm.at[pl.ds(kk * _EPAD + base, _C)], srcidx_v)
            pltpu.sync_copy(dst_hbm.at[pl.ds(kk * _EPAD + base, _C)], dstidx_v)
            pltpu.async_copy(ssrc_hbm.at[srcidx_v], ssrcg_v, sem).wait()

        def per_worker(r, _):
            w = r * 32 + wid
            d0 = pl.multiple_of(w * rows, 128)
            def zrow(i, _c):
                out_v[pl.ds(pl.multiple_of(i * 16, 16), 16)] = zero16
                return _c
            lax.fori_loop(0, rows * _H // 16, zrow, 0)

            for kk in range(len(k_rels)):
                evec = offs_s[kk, pl.ds(w, 16)]
                e0 = evec[0]
                e1 = evec[1]
                a0 = (e0 >> 7) << 7
                nch = (e1 - a0 + _C - 1) // _C
                pltpu.sync_copy(sdst_hbm.at[pl.ds(pl.multiple_of(kk * ndp + d0, 128), rows)], sdst_v)
                def zden(i, _c):
                    den_v[pl.ds(pl.multiple_of(i * 16, 16), 16)] = zero16
                    return _c
                lax.fori_loop(0, rows // 16, zden, 0)

                def p1(c, _c):
                    base = pl.multiple_of(a0 + c * _C, 128)
                    load_chunk(kk, base)
                    for j in range(_C // 16):
                        dl16, ee = edge_scalars(kk, base, d0, e0, e1, j)
                        ee_v[pl.ds(j * 16, 16)] = ee
                        dl_v[pl.ds(j * 16, 16)] = dl16
                    def acc(i, _a):
                        bidx = jnp.full((16,), i, jnp.int32)
                        dlb = plsc.load_gather(dl_v, [bidx])
                        eeb = plsc.load_gather(ee_v, [bidx])
                        plsc.addupdate_scatter(den_v, [dlb], eeb, mask=lane0)
                        return _a
                    lax.fori_loop(0, _C, acc, 0)
                    return _c
                lax.fori_loop(0, nch, p1, 0)

                def p2(c, _c):
                    base = pl.multiple_of(a0 + c * _C, 128)
                    load_chunk(kk, base)
                    pltpu.async_copy(hs_hbm.at[srcidx_v], rows_v, sem).wait()
                    for j in range(_C // 16):
                        dl16, ee = edge_scalars(kk, base, d0, e0, e1, j)
                        den16 = plsc.load_gather(den_v, [dl16])
                        alpha = ee / (den16 + 1e-16)
                        ee_v[pl.ds(j * 16, 16)] = alpha
                        dl_v[pl.ds(j * 16, 16)] = dl16
                    def acc(i, _a):
                        bidx = jnp.full((16,), i, jnp.int32)
                        dlb = plsc.load_gather(dl_v, [bidx])
                        ab = plsc.load_gather(ee_v, [bidx])
                        obase = dlb * _H + lanes
                        for j in range(_H // 16):
                            vals = rows_v[i, pl.ds(j * 16, 16)] * ab
                            plsc.addupdate_scatter(out_v, [obase + j * 16], vals)
                        return _a
                    lax.fori_loop(0, _C, acc, 0)
                    return _c
                lax.fori_loop(0, nch, p2, 0)

            pltpu.sync_copy(out_v, out_hbm.at[pl.ds(d0 * _H, rows * _H)])
            return _

        lax.fori_loop(0, nrounds, per_worker, 0)

    return sc_kernel


def kernel(x_variable, x_value, x_operator, x_constraint, edge_index_0, edge_index_1,
           edge_index_2, edge_index_3, edge_index_4, edge_index_5, edge_index_6,
           edge_index_7, edge_index_8, batch_variable, batch_constraint,
           Wsrc, Wdst, att_src, att_dst, bias, linW, linb):
    xs = {'variable': x_variable, 'value': x_value, 'operator': x_operator,
          'constraint': x_constraint}
    edges = [edge_index_0, edge_index_1, edge_index_2, edge_index_3, edge_index_4,
             edge_index_5, edge_index_6, edge_index_7, edge_index_8]

    src_rels = {t: [r for r, (s, d) in enumerate(_RELS) if s == t] for t in _TYPES}
    dst_rels = {t: [r for r, (s, d) in enumerate(_RELS) if d == t] for t in _TYPES}

    srt = {}
    for dtp, rels in _GROUPS.items():
        w_workers, rows = _WCFG[dtp]
        op_cols = w_workers + 16
        base = 0
        src_list, dst_list, off_list, bases = [], [], [], []
        for r in rels:
            src, dst = edges[r][0].astype(jnp.int32), edges[r][1].astype(jnp.int32)
            order = jnp.argsort(dst)
            dst_s = dst[order]
            src_s = src[order] + base
            offs = jnp.searchsorted(dst_s, jnp.arange(w_workers + 1, dtype=jnp.int32) * rows
                                    ).astype(jnp.int32)
            offs = jnp.pad(offs, (0, op_cols - (w_workers + 1)), constant_values=_E)
            src_list.append(jnp.pad(src_s, (0, _EPAD - _E)))
            dst_list.append(jnp.pad(dst_s, (0, _EPAD - _E), constant_values=w_workers * rows))
            off_list.append(offs)
            bases.append(base)
            base += _NT[_RELS[r][0]]
        srt[dtp] = (jnp.concatenate(src_list), jnp.concatenate(dst_list),
                    jnp.stack(off_list), bases)

    sc_kernels = {dtp: _make_sc_kernel(rels, _WCFG[dtp][0], _WCFG[dtp][1],
                                       None, _WCFG[dtp][0] + 16)
                  for dtp, rels in _GROUPS.items()}

    x = {t: v for t, v in xs.items()}
    for l in range(_L):
        hs = {}
        ssrc = {}
        sdst = {}
        for t in _TYPES:
            n = _NT[t]
            npad = _round_up(n, 512)
            cols = []
            for r in src_rels[t]:
                cols.append(Wsrc[l, r])
            for r in src_rels[t]:
                cols.append((Wsrc[l, r] @ att_src[l, r])[:, None])
            for r in dst_rels[t]:
                cols.append((Wdst[l, r] @ att_dst[l, r])[:, None])
            wcat = jnp.concatenate(cols, axis=1)
            cp = _round_up(wcat.shape[1], _H)
            wcat = jnp.pad(wcat, ((0, 0), (0, cp - wcat.shape[1])))
            xp = jnp.pad(x[t], ((0, npad - x[t].shape[0]), (0, 0)))
            y = _matmul(xp, wcat)
            ks = len(src_rels[t])
            for i, r in enumerate(src_rels[t]):
                hs[r] = y[:n, i * _H:(i + 1) * _H]
                ssrc[r] = y[:n, ks * _H + i]
            for j, r in enumerate(dst_rels[t]):
                sdst[r] = y[:n, ks * _H + ks + j]

        acc = {}
        for dtp, rels in _GROUPS.items():
            w_workers, rows = _WCFG[dtp]
            ndp = w_workers * rows
            src_st, dst_st, off_st, bases = srt[dtp]
            hs_cat = jnp.concatenate([hs[r] for r in rels], axis=0)
            ssrc_cat = jnp.concatenate([ssrc[r] for r in rels], axis=0)
            sdst_all = jnp.concatenate([jnp.pad(sdst[r], (0, ndp - _NT[dtp])) for r in rels])
            m_k = []
            for r in rels:
                s = jnp.max(ssrc[r]) + jnp.max(sdst[r])
                m_k.append(jnp.where(s > 0, s, 0.2 * s))
            m_k = jnp.stack(m_k)
            m_k = jnp.pad(m_k, (0, 16 - len(rels)))
            out = sc_kernels[dtp](hs_cat, ssrc_cat, sdst_all, src_st, dst_st,
                                  off_st, m_k)
            out = out.reshape(ndp, _H)
            bsum = sum(bias[l, r] for r in rels)
            acc[dtp] = out[:_NT[dtp]] + bsum[None, :]
        x = {t: jax.nn.relu(v) for t, v in acc.items()}

    sv = _pool_sums(x['variable'], batch_variable)
    sc_ = _pool_sums(x['constraint'], batch_constraint)
    return _head(sv, sc_, linW, linb)


# speedup vs baseline: 2.9078x; 1.1195x over previous
import functools
import jax
import jax.numpy as jnp
from jax import lax
from jax.experimental import pallas as pl
from jax.experimental.pallas import tpu as pltpu
from jax.experimental.pallas import tpu_sc as plsc

_RELS = [('variable', 'value'), ('variable', 'operator'), ('variable', 'constraint'),
         ('operator', 'constraint'), ('constraint', 'constraint'), ('value', 'variable'),
         ('operator', 'variable'), ('constraint', 'variable'), ('constraint', 'operator')]
_NT = {'variable': 50000, 'value': 2000, 'operator': 500, 'constraint': 10000}
_TYPES = ['variable', 'value', 'operator', 'constraint']
_B = 64
_L = 2
_E = 64000
_H = 128
_C = 128
_EPAD = _E + _C

_GROUPS = {'value': [0], 'operator': [1, 8], 'constraint': [2, 3, 4], 'variable': [5, 6, 7]}
_WCFG = {'variable': (128, 512), 'constraint': (32, 384), 'value': (32, 128), 'operator': (32, 128)}


def _round_up(n, m):
    return (n + m - 1) // m * m


def _mm_body(x_ref, w_ref, o_ref):
    o_ref[...] = jnp.dot(x_ref[...], w_ref[...], preferred_element_type=jnp.float32)


def _matmul(x, w):
    npad, cp = x.shape[0], w.shape[1]
    nb = npad // 512
    return pl.pallas_call(
        _mm_body,
        grid=(nb,),
        in_specs=[pl.BlockSpec((512, _H), lambda i: (i, 0)),
                  pl.BlockSpec((_H, cp), lambda i: (0, 0))],
        out_specs=pl.BlockSpec((512, cp), lambda i: (i, 0)),
        out_shape=jax.ShapeDtypeStruct((npad, cp), jnp.float32),
    )(x, w)


def _pool_body(ids_ref, x_ref, o_ref):
    @pl.when(pl.program_id(0) == 0)
    def _():
        o_ref[...] = jnp.zeros_like(o_ref)
    ids = ids_ref[0, 0, :]
    mask = (ids[None, :] == lax.broadcasted_iota(jnp.int32, (_B, 512), 0)).astype(jnp.float32)
    o_ref[...] += jnp.dot(mask, x_ref[...], preferred_element_type=jnp.float32)


def _pool_sums(x, ids):
    n = x.shape[0]
    npad = _round_up(n, 512)
    xp = jnp.concatenate([x, jnp.ones((n, 1), jnp.float32)], axis=1)
    xp = jnp.pad(xp, ((0, npad - n), (0, 256 - _H - 1)))
    idp = jnp.pad(ids.astype(jnp.int32), (0, npad - n), constant_values=_B)
    idp = idp.reshape(npad // 512, 1, 512)
    nb = npad // 512
    return pl.pallas_call(
        _pool_body,
        grid=(nb,),
        in_specs=[pl.BlockSpec((1, 1, 512), lambda i: (i, 0, 0)),
                  pl.BlockSpec((512, 256), lambda i: (i, 0))],
        out_specs=pl.BlockSpec((_B, 256), lambda i: (0, 0)),
        out_shape=jax.ShapeDtypeStruct((_B, 256), jnp.float32),
    )(idp, xp)


def _head_body(sv_ref, sc_ref, w_ref, b_ref, o_ref):
    sv = sv_ref[...]
    sc = sc_ref[...]
    cv = jnp.maximum(sv[:, _H:_H + 1], 1.0)
    cc = jnp.maximum(sc[:, _H:_H + 1], 1.0)
    pooled = jnp.concatenate([sv[:, :_H] / cv, sc[:, :_H] / cc], axis=1)
    o_ref[...] = jnp.dot(pooled, w_ref[...], preferred_element_type=jnp.float32) + b_ref[...]


def _head(sv, sc, lin_w, lin_b):
    wp = jnp.pad(lin_w, ((0, 0), (0, _H - lin_w.shape[1])))
    bp = jnp.pad(lin_b, (0, _H - lin_b.shape[0]))[None, :]
    out = pl.pallas_call(
        _head_body,
        out_shape=jax.ShapeDtypeStruct((_B, _H), jnp.float32),
    )(sv, sc, wp, bp)
    return out[:, :lin_w.shape[1]]


def _make_sc_kernel(k_rels, w_workers, rows, ns_cat, op_cols):
    mesh = plsc.VectorSubcoreMesh(core_axis_name="c", subcore_axis_name="s")
    nrounds = w_workers // 32
    ndp = w_workers * rows

    @functools.partial(
        pl.kernel,
        mesh=mesh,
        compiler_params=pltpu.CompilerParams(needs_layout_passes=False),
        out_type=jax.ShapeDtypeStruct((ndp * _H,), jnp.float32),
        scratch_types=[
            pltpu.VMEM((rows * _H,), jnp.float32),
            pltpu.VMEM((_C, _H), jnp.float32),
            pltpu.VMEM((_C,), jnp.int32),
            pltpu.VMEM((_C,), jnp.int32),
            pltpu.VMEM((_C,), jnp.float32),
            pltpu.VMEM((_C,), jnp.float32),
            pltpu.VMEM((_C,), jnp.int32),
            pltpu.VMEM((rows,), jnp.float32),
            pltpu.VMEM((rows,), jnp.float32),
            pltpu.VMEM((len(k_rels), op_cols), jnp.int32),
            pltpu.VMEM((16,), jnp.float32),
            pltpu.SemaphoreType.DMA,
        ],
    )
    def sc_kernel(hs_hbm, ssrc_hbm, sdst_hbm, src_hbm, dst_hbm, offs_hbm, m_hbm,
                  out_hbm,
                  out_v, rows_v, srcidx_v, dstidx_v, ssrcg_v, ee_v, dl_v,
                  sdst_v, den_v, offs_s, m_s, sem):
        wid = lax.axis_index("s") * 2 + lax.axis_index("c")
        pltpu.sync_copy(offs_hbm, offs_s)
        pltpu.sync_copy(m_hbm, m_s)
        lanes = lax.iota(jnp.int32, 16)
        lane0 = lanes == 0
        zero16 = jnp.zeros((16,), jnp.float32)
        mvec = m_s[...]

        def edge_scalars(kk, base, d0, e0, e1, j):
            gidx = base + j * 16 + lanes
            valid = (gidx >= e0) & (gidx < e1)
            dst16 = dstidx_v[pl.ds(j * 16, 16)]
            dl16 = jnp.clip(dst16 - d0, 0, rows - 1)
            ssrc16 = ssrcg_v[pl.ds(j * 16, 16)]
            sdst16 = plsc.load_gather(sdst_v, [dl16])
            s = ssrc16 + sdst16
            e = jnp.where(s > 0, s, 0.2 * s)
            ee = jnp.exp(e - mvec[kk])
            ee = jnp.where(valid, ee, 0.0)
            return dl16, ee

        def load_idx(kk, base):
            c1 = pltpu.async_copy(src_hbm.at[pl.ds(kk * _EPAD + base, _C)], srcidx_v, sem)
            c2 = pltpu.async_copy(dst_hbm.at[pl.ds(kk * _EPAD + base, _C)], dstidx_v, sem)
            c1.wait()
            c2.wait()

        def per_worker(r, _):
            w = r * 32 + wid
            d0 = pl.multiple_of(w * rows, 128)
            def zrow(i, _c):
                out_v[pl.ds(pl.multiple_of(i * 16, 16), 16)] = zero16
                return _c
            lax.fori_loop(0, rows * _H // 16, zrow, 0)

            for kk in range(len(k_rels)):
                evec = offs_s[kk, pl.ds(w, 16)]
                e0 = evec[0]
                e1 = evec[1]
                a0 = (e0 >> 7) << 7
                nch = (e1 - a0 + _C - 1) // _C
                pltpu.sync_copy(sdst_hbm.at[pl.ds(pl.multiple_of(kk * ndp + d0, 128), rows)], sdst_v)
                def zden(i, _c):
                    den_v[pl.ds(pl.multiple_of(i * 16, 16), 16)] = zero16
                    return _c
                lax.fori_loop(0, rows // 16, zden, 0)

                def p1(c, _c):
                    base = pl.multiple_of(a0 + c * _C, 128)
                    load_idx(kk, base)
                    pltpu.async_copy(ssrc_hbm.at[srcidx_v], ssrcg_v, sem).wait()
                    for j in range(_C // 16):
                        dl16, ee = edge_scalars(kk, base, d0, e0, e1, j)
                        ee_v[pl.ds(j * 16, 16)] = ee
                        dl_v[pl.ds(j * 16, 16)] = dl16
                    def acc(i, _a):
                        bidx = jnp.full((16,), i, jnp.int32)
                        dlb = plsc.load_gather(dl_v, [bidx])
                        eeb = plsc.load_gather(ee_v, [bidx])
                        plsc.addupdate_scatter(den_v, [dlb], eeb, mask=lane0)
                        return _a
                    lax.fori_loop(0, _C, acc, 0)
                    return _c
                lax.fori_loop(0, nch, p1, 0)

                def p2(c, _c):
                    base = pl.multiple_of(a0 + c * _C, 128)
                    load_idx(kk, base)
                    g1 = pltpu.async_copy(ssrc_hbm.at[srcidx_v], ssrcg_v, sem)
                    g2 = pltpu.async_copy(hs_hbm.at[srcidx_v], rows_v, sem)
                    g1.wait()
                    g2.wait()
                    for j in range(_C // 16):
                        dl16, ee = edge_scalars(kk, base, d0, e0, e1, j)
                        den16 = plsc.load_gather(den_v, [dl16])
                        alpha = ee / (den16 + 1e-16)
                        ee_v[pl.ds(j * 16, 16)] = alpha
                        dl_v[pl.ds(j * 16, 16)] = dl16
                    def acc(i, _a):
                        bidx = jnp.full((16,), i, jnp.int32)
                        dlb = plsc.load_gather(dl_v, [bidx])
                        ab = plsc.load_gather(ee_v, [bidx])
                        obase = dlb * _H + lanes
                        for j in range(_H // 16):
                            vals = rows_v[i, pl.ds(j * 16, 16)] * ab
                            plsc.addupdate_scatter(out_v, [obase + j * 16], vals)
                        return _a
                    lax.fori_loop(0, _C, acc, 0)
                    return _c
                lax.fori_loop(0, nch, p2, 0)

            pltpu.sync_copy(out_v, out_hbm.at[pl.ds(d0 * _H, rows * _H)])
            return _

        lax.fori_loop(0, nrounds, per_worker, 0)

    return sc_kernel


def kernel(x_variable, x_value, x_operator, x_constraint, edge_index_0, edge_index_1,
           edge_index_2, edge_index_3, edge_index_4, edge_index_5, edge_index_6,
           edge_index_7, edge_index_8, batch_variable, batch_constraint,
           Wsrc, Wdst, att_src, att_dst, bias, linW, linb):
    xs = {'variable': x_variable, 'value': x_value, 'operator': x_operator,
          'constraint': x_constraint}
    edges = [edge_index_0, edge_index_1, edge_index_2, edge_index_3, edge_index_4,
             edge_index_5, edge_index_6, edge_index_7, edge_index_8]

    src_rels = {t: [r for r, (s, d) in enumerate(_RELS) if s == t] for t in _TYPES}
    dst_rels = {t: [r for r, (s, d) in enumerate(_RELS) if d == t] for t in _TYPES}

    srt = {}
    for dtp, rels in _GROUPS.items():
        w_workers, rows = _WCFG[dtp]
        op_cols = w_workers + 16
        base = 0
        src_list, dst_list, off_list, bases = [], [], [], []
        for r in rels:
            src, dst = edges[r][0].astype(jnp.int32), edges[r][1].astype(jnp.int32)
            order = jnp.argsort(dst)
            dst_s = dst[order]
            src_s = src[order] + base
            offs = jnp.searchsorted(dst_s, jnp.arange(w_workers + 1, dtype=jnp.int32) * rows
                                    ).astype(jnp.int32)
            offs = jnp.pad(offs, (0, op_cols - (w_workers + 1)), constant_values=_E)
            src_list.append(jnp.pad(src_s, (0, _EPAD - _E)))
            dst_list.append(jnp.pad(dst_s, (0, _EPAD - _E), constant_values=w_workers * rows))
            off_list.append(offs)
            bases.append(base)
            base += _NT[_RELS[r][0]]
        srt[dtp] = (jnp.concatenate(src_list), jnp.concatenate(dst_list),
                    jnp.stack(off_list), bases)

    sc_kernels = {dtp: _make_sc_kernel(rels, _WCFG[dtp][0], _WCFG[dtp][1],
                                       None, _WCFG[dtp][0] + 16)
                  for dtp, rels in _GROUPS.items()}

    x = {t: v for t, v in xs.items()}
    for l in range(_L):
        hs = {}
        ssrc = {}
        sdst = {}
        for t in _TYPES:
            n = _NT[t]
            npad = _round_up(n, 512)
            cols = []
            for r in src_rels[t]:
                cols.append(Wsrc[l, r])
            for r in src_rels[t]:
                cols.append((Wsrc[l, r] @ att_src[l, r])[:, None])
            for r in dst_rels[t]:
                cols.append((Wdst[l, r] @ att_dst[l, r])[:, None])
            wcat = jnp.concatenate(cols, axis=1)
            cp = _round_up(wcat.shape[1], _H)
            wcat = jnp.pad(wcat, ((0, 0), (0, cp - wcat.shape[1])))
            xp = jnp.pad(x[t], ((0, npad - x[t].shape[0]), (0, 0)))
            y = _matmul(xp, wcat)
            ks = len(src_rels[t])
            for i, r in enumerate(src_rels[t]):
                hs[r] = y[:n, i * _H:(i + 1) * _H]
                ssrc[r] = y[:n, ks * _H + i]
            for j, r in enumerate(dst_rels[t]):
                sdst[r] = y[:n, ks * _H + ks + j]

        acc = {}
        for dtp, rels in _GROUPS.items():
            w_workers, rows = _WCFG[dtp]
            ndp = w_workers * rows
            src_st, dst_st, off_st, bases = srt[dtp]
            hs_cat = jnp.concatenate([hs[r] for r in rels], axis=0)
            ssrc_cat = jnp.concatenate([ssrc[r] for r in rels], axis=0)
            sdst_all = jnp.concatenate([jnp.pad(sdst[r], (0, ndp - _NT[dtp])) for r in rels])
            m_k = []
            for r in rels:
                s = jnp.max(ssrc[r]) + jnp.max(sdst[r])
                m_k.append(jnp.where(s > 0, s, 0.2 * s))
            m_k = jnp.stack(m_k)
            m_k = jnp.pad(m_k, (0, 16 - len(rels)))
            out = sc_kernels[dtp](hs_cat, ssrc_cat, sdst_all, src_st, dst_st,
                                  off_st, m_k)
            out = out.reshape(ndp, _H)
            bsum = sum(bias[l, r] for r in rels)
            acc[dtp] = out[:_NT[dtp]] + bsum[None, :]
        x = {t: jax.nn.relu(v) for t, v in acc.items()}

    sv = _pool_sums(x['variable'], batch_variable)
    sc_ = _pool_sums(x['constraint'], batch_constraint)
    return _head(sv, sc_, linW, linb)


# speedup vs baseline: 4.7130x; 1.6208x over previous
import functools
import jax
import jax.numpy as jnp
from jax import lax
from jax.experimental import pallas as pl
from jax.experimental.pallas import tpu as pltpu
from jax.experimental.pallas import tpu_sc as plsc

_RELS = [('variable', 'value'), ('variable', 'operator'), ('variable', 'constraint'),
         ('operator', 'constraint'), ('constraint', 'constraint'), ('value', 'variable'),
         ('operator', 'variable'), ('constraint', 'variable'), ('constraint', 'operator')]
_NT = {'variable': 50000, 'value': 2000, 'operator': 500, 'constraint': 10000}
_TYPES = ['variable', 'value', 'operator', 'constraint']
_B = 64
_L = 2
_E = 64000
_H = 128
_C = 128
_EPAD = _E + _C

_GROUPS = {'value': [0], 'operator': [1, 8], 'constraint': [2, 3, 4], 'variable': [5, 6, 7]}
_WCFG = {'variable': (128, 400), 'constraint': (32, 320), 'value': (32, 64), 'operator': (32, 16)}


def _round_up(n, m):
    return (n + m - 1) // m * m


def _mm_body(x_ref, w_ref, o_ref):
    o_ref[...] = jnp.dot(x_ref[...], w_ref[...], preferred_element_type=jnp.float32)


def _matmul(x, w):
    npad, cp = x.shape[0], w.shape[1]
    nb = npad // 512
    return pl.pallas_call(
        _mm_body,
        grid=(nb,),
        in_specs=[pl.BlockSpec((512, _H), lambda i: (i, 0)),
                  pl.BlockSpec((_H, cp), lambda i: (0, 0))],
        out_specs=pl.BlockSpec((512, cp), lambda i: (i, 0)),
        out_shape=jax.ShapeDtypeStruct((npad, cp), jnp.float32),
    )(x, w)


def _pool_body(ids_ref, x_ref, o_ref):
    @pl.when(pl.program_id(0) == 0)
    def _():
        o_ref[...] = jnp.zeros_like(o_ref)
    ids = ids_ref[0, 0, :]
    mask = (ids[None, :] == lax.broadcasted_iota(jnp.int32, (_B, 512), 0)).astype(jnp.float32)
    o_ref[...] += jnp.dot(mask, x_ref[...], preferred_element_type=jnp.float32)


def _pool_sums(x, ids):
    n = x.shape[0]
    npad = _round_up(n, 512)
    xp = jnp.concatenate([x, jnp.ones((n, 1), jnp.float32)], axis=1)
    xp = jnp.pad(xp, ((0, npad - n), (0, 256 - _H - 1)))
    idp = jnp.pad(ids.astype(jnp.int32), (0, npad - n), constant_values=_B)
    idp = idp.reshape(npad // 512, 1, 512)
    nb = npad // 512
    return pl.pallas_call(
        _pool_body,
        grid=(nb,),
        in_specs=[pl.BlockSpec((1, 1, 512), lambda i: (i, 0, 0)),
                  pl.BlockSpec((512, 256), lambda i: (i, 0))],
        out_specs=pl.BlockSpec((_B, 256), lambda i: (0, 0)),
        out_shape=jax.ShapeDtypeStruct((_B, 256), jnp.float32),
    )(idp, xp)


def _head_body(sv_ref, sc_ref, w_ref, b_ref, o_ref):
    sv = sv_ref[...]
    sc = sc_ref[...]
    cv = jnp.maximum(sv[:, _H:_H + 1], 1.0)
    cc = jnp.maximum(sc[:, _H:_H + 1], 1.0)
    pooled = jnp.concatenate([sv[:, :_H] / cv, sc[:, :_H] / cc], axis=1)
    o_ref[...] = jnp.dot(pooled, w_ref[...], preferred_element_type=jnp.float32) + b_ref[...]


def _head(sv, sc, lin_w, lin_b):
    wp = jnp.pad(lin_w, ((0, 0), (0, _H - lin_w.shape[1])))
    bp = jnp.pad(lin_b, (0, _H - lin_b.shape[0]))[None, :]
    out = pl.pallas_call(
        _head_body,
        out_shape=jax.ShapeDtypeStruct((_B, _H), jnp.float32),
    )(sv, sc, wp, bp)
    return out[:, :lin_w.shape[1]]


def _make_sc_kernel(k_rels, w_workers, rows, ns_cat, op_cols):
    mesh = plsc.VectorSubcoreMesh(core_axis_name="c", subcore_axis_name="s")
    nrounds = w_workers // 32
    ndp = w_workers * rows

    @functools.partial(
        pl.kernel,
        mesh=mesh,
        compiler_params=pltpu.CompilerParams(needs_layout_passes=False),
        out_type=jax.ShapeDtypeStruct((ndp * _H,), jnp.float32),
        scratch_types=[
            pltpu.VMEM((rows * _H,), jnp.float32),
            pltpu.VMEM((_C, _H), jnp.float32),
            pltpu.VMEM((_C,), jnp.int32),
            pltpu.VMEM((_C,), jnp.int32),
            pltpu.VMEM((_C,), jnp.float32),
            pltpu.VMEM((_C,), jnp.float32),
            pltpu.VMEM((_C,), jnp.int32),
            pltpu.VMEM((rows + 128,), jnp.float32),
            pltpu.VMEM((rows,), jnp.float32),
            pltpu.VMEM((len(k_rels), op_cols), jnp.int32),
            pltpu.VMEM((16,), jnp.float32),
            pltpu.SemaphoreType.DMA,
        ],
    )
    def sc_kernel(hs_hbm, ssrc_hbm, sdst_hbm, src_hbm, dst_hbm, offs_hbm, m_hbm,
                  out_hbm,
                  out_v, rows_v, srcidx_v, dstidx_v, ssrcg_v, ee_v, dl_v,
                  sdst_v, den_v, offs_s, m_s, sem):
        wid = lax.axis_index("s") * 2 + lax.axis_index("c")
        pltpu.sync_copy(offs_hbm, offs_s)
        pltpu.sync_copy(m_hbm, m_s)
        lanes = lax.iota(jnp.int32, 16)
        lane0 = lanes == 0
        zero16 = jnp.zeros((16,), jnp.float32)
        mvec = m_s[...]

        def edge_scalars(kk, base, d0, doff, e0, e1, j):
            gidx = base + j * 16 + lanes
            valid = (gidx >= e0) & (gidx < e1)
            dst16 = dstidx_v[pl.ds(j * 16, 16)]
            dl16 = jnp.clip(dst16 - d0, 0, rows - 1)
            ssrc16 = ssrcg_v[pl.ds(j * 16, 16)]
            sdst16 = plsc.load_gather(sdst_v, [dl16 + doff])
            s = ssrc16 + sdst16
            e = jnp.where(s > 0, s, 0.2 * s)
            ee = jnp.exp(e - mvec[kk])
            ee = jnp.where(valid, ee, 0.0)
            return dl16, ee

        def load_idx(kk, base):
            c1 = pltpu.async_copy(src_hbm.at[pl.ds(kk * _EPAD + base, _C)], srcidx_v, sem)
            c2 = pltpu.async_copy(dst_hbm.at[pl.ds(kk * _EPAD + base, _C)], dstidx_v, sem)
            c1.wait()
            c2.wait()

        def per_worker(r, _):
            w = r * 32 + wid
            d0 = w * rows
            d0a = pl.multiple_of((d0 >> 7) << 7, 128)
            doff = d0 - d0a
            def zrow(i, _c):
                out_v[pl.ds(pl.multiple_of(i * 16, 16), 16)] = zero16
                return _c
            lax.fori_loop(0, rows * _H // 16, zrow, 0)

            for kk in range(len(k_rels)):
                evec = offs_s[kk, pl.ds(w, 16)]
                e0 = evec[0]
                e1 = evec[1]
                a0 = (e0 >> 7) << 7
                nch = (e1 - a0 + _C - 1) // _C
                pltpu.sync_copy(sdst_hbm.at[pl.ds(pl.multiple_of(kk * ndp + d0a, 128), rows + 128)], sdst_v)
                def zden(i, _c):
                    den_v[pl.ds(pl.multiple_of(i * 16, 16), 16)] = zero16
                    return _c
                lax.fori_loop(0, rows // 16, zden, 0)

                def p1(c, _c):
                    base = pl.multiple_of(a0 + c * _C, 128)
                    load_idx(kk, base)
                    pltpu.async_copy(ssrc_hbm.at[srcidx_v], ssrcg_v, sem).wait()
                    for j in range(_C // 16):
                        dl16, ee = edge_scalars(kk, base, d0, doff, e0, e1, j)
                        ee_v[pl.ds(j * 16, 16)] = ee
                        dl_v[pl.ds(j * 16, 16)] = dl16
                    def acc(i, _a):
                        bidx = jnp.full((16,), i, jnp.int32)
                        dlb = plsc.load_gather(dl_v, [bidx])
                        eeb = plsc.load_gather(ee_v, [bidx])
                        plsc.addupdate_scatter(den_v, [dlb], eeb, mask=lane0)
                        return _a
                    lax.fori_loop(0, _C, acc, 0)
                    return _c
                lax.fori_loop(0, nch, p1, 0)

                def p2(c, _c):
                    base = pl.multiple_of(a0 + c * _C, 128)
                    load_idx(kk, base)
                    g1 = pltpu.async_copy(ssrc_hbm.at[srcidx_v], ssrcg_v, sem)
                    g2 = pltpu.async_copy(hs_hbm.at[srcidx_v], rows_v, sem)
                    g1.wait()
                    g2.wait()
                    for j in range(_C // 16):
                        dl16, ee = edge_scalars(kk, base, d0, doff, e0, e1, j)
                        den16 = plsc.load_gather(den_v, [dl16])
                        alpha = ee / (den16 + 1e-16)
                        ee_v[pl.ds(j * 16, 16)] = alpha
                        dl_v[pl.ds(j * 16, 16)] = dl16
                    def acc(i, _a):
                        bidx = jnp.full((16,), i, jnp.int32)
                        dlb = plsc.load_gather(dl_v, [bidx])
                        ab = plsc.load_gather(ee_v, [bidx])
                        obase = dlb * _H + lanes
                        for j in range(_H // 16):
                            vals = rows_v[i, pl.ds(j * 16, 16)] * ab
                            plsc.addupdate_scatter(out_v, [obase + j * 16], vals)
                        return _a
                    lax.fori_loop(0, _C, acc, 0)
                    return _c
                lax.fori_loop(0, nch, p2, 0)

            pltpu.sync_copy(out_v, out_hbm.at[pl.ds(pl.multiple_of(d0 * _H, 128), rows * _H)])
            return _

        lax.fori_loop(0, nrounds, per_worker, 0)

    return sc_kernel


def kernel(x_variable, x_value, x_operator, x_constraint, edge_index_0, edge_index_1,
           edge_index_2, edge_index_3, edge_index_4, edge_index_5, edge_index_6,
           edge_index_7, edge_index_8, batch_variable, batch_constraint,
           Wsrc, Wdst, att_src, att_dst, bias, linW, linb):
    xs = {'variable': x_variable, 'value': x_value, 'operator': x_operator,
          'constraint': x_constraint}
    edges = [edge_index_0, edge_index_1, edge_index_2, edge_index_3, edge_index_4,
             edge_index_5, edge_index_6, edge_index_7, edge_index_8]

    src_rels = {t: [r for r, (s, d) in enumerate(_RELS) if s == t] for t in _TYPES}
    dst_rels = {t: [r for r, (s, d) in enumerate(_RELS) if d == t] for t in _TYPES}

    srt = {}
    for dtp, rels in _GROUPS.items():
        w_workers, rows = _WCFG[dtp]
        op_cols = w_workers + 16
        base = 0
        src_list, dst_list, off_list, bases = [], [], [], []
        for r in rels:
            src, dst = edges[r][0].astype(jnp.int32), edges[r][1].astype(jnp.int32)
            order = jnp.argsort(dst)
            dst_s = dst[order]
            src_s = src[order] + base
            offs = jnp.searchsorted(dst_s, jnp.arange(w_workers + 1, dtype=jnp.int32) * rows
                                    ).astype(jnp.int32)
            offs = jnp.pad(offs, (0, op_cols - (w_workers + 1)), constant_values=_E)
            src_list.append(jnp.pad(src_s, (0, _EPAD - _E)))
            dst_list.append(jnp.pad(dst_s, (0, _EPAD - _E), constant_values=w_workers * rows))
            off_list.append(offs)
            bases.append(base)
            base += _NT[_RELS[r][0]]
        srt[dtp] = (jnp.concatenate(src_list), jnp.concatenate(dst_list),
                    jnp.stack(off_list), bases)

    sc_kernels = {dtp: _make_sc_kernel(rels, _WCFG[dtp][0], _WCFG[dtp][1],
                                       None, _WCFG[dtp][0] + 16)
                  for dtp, rels in _GROUPS.items()}

    x = {t: v for t, v in xs.items()}
    for l in range(_L):
        hs = {}
        ssrc = {}
        sdst = {}
        for t in _TYPES:
            n = _NT[t]
            npad = _round_up(n, 512)
            cols = []
            for r in src_rels[t]:
                cols.append(Wsrc[l, r])
            for r in src_rels[t]:
                cols.append((Wsrc[l, r] @ att_src[l, r])[:, None])
            for r in dst_rels[t]:
                cols.append((Wdst[l, r] @ att_dst[l, r])[:, None])
            wcat = jnp.concatenate(cols, axis=1)
            cp = _round_up(wcat.shape[1], _H)
            wcat = jnp.pad(wcat, ((0, 0), (0, cp - wcat.shape[1])))
            xp = jnp.pad(x[t], ((0, npad - x[t].shape[0]), (0, 0)))
            y = _matmul(xp, wcat)
            ks = len(src_rels[t])
            for i, r in enumerate(src_rels[t]):
                hs[r] = y[:n, i * _H:(i + 1) * _H]
                ssrc[r] = y[:n, ks * _H + i]
            for j, r in enumerate(dst_rels[t]):
                sdst[r] = y[:n, ks * _H + ks + j]

        acc = {}
        for dtp, rels in _GROUPS.items():
            w_workers, rows = _WCFG[dtp]
            ndp = w_workers * rows
            src_st, dst_st, off_st, bases = srt[dtp]
            hs_cat = jnp.concatenate([hs[r] for r in rels], axis=0)
            ssrc_cat = jnp.concatenate([ssrc[r] for r in rels], axis=0)
            sdst_all = jnp.concatenate([jnp.pad(sdst[r], (0, ndp - _NT[dtp])) for r in rels]
                                       + [jnp.zeros((128,), jnp.float32)])
            m_k = []
            for r in rels:
                s = jnp.max(ssrc[r]) + jnp.max(sdst[r])
                m_k.append(jnp.where(s > 0, s, 0.2 * s))
            m_k = jnp.stack(m_k)
            m_k = jnp.pad(m_k, (0, 16 - len(rels)))
            out = sc_kernels[dtp](hs_cat, ssrc_cat, sdst_all, src_st, dst_st,
                                  off_st, m_k)
            out = out.reshape(ndp, _H)
            bsum = sum(bias[l, r] for r in rels)
            acc[dtp] = out[:_NT[dtp]] + bsum[None, :]
        x = {t: jax.nn.relu(v) for t, v in acc.items()}

    sv = _pool_sums(x['variable'], batch_variable)
    sc_ = _pool_sums(x['constraint'], batch_constraint)
    return _head(sv, sc_, linW, linb)


# speedup vs baseline: 5.1776x; 1.0986x over previous
import functools
import jax
import jax.numpy as jnp
from jax import lax
from jax.experimental import pallas as pl
from jax.experimental.pallas import tpu as pltpu
from jax.experimental.pallas import tpu_sc as plsc

_RELS = [('variable', 'value'), ('variable', 'operator'), ('variable', 'constraint'),
         ('operator', 'constraint'), ('constraint', 'constraint'), ('value', 'variable'),
         ('operator', 'variable'), ('constraint', 'variable'), ('constraint', 'operator')]
_NT = {'variable': 50000, 'value': 2000, 'operator': 500, 'constraint': 10000}
_TYPES = ['variable', 'value', 'operator', 'constraint']
_B = 64
_L = 2
_E = 64000
_H = 128
_C = 128
_EPAD = _E + 512

_GROUPS = {'value': [0], 'operator': [1, 8], 'constraint': [2, 3, 4], 'variable': [5, 6, 7]}
_WCFG = {'variable': (128, 400), 'constraint': (32, 320), 'value': (32, 64), 'operator': (32, 16)}


def _round_up(n, m):
    return (n + m - 1) // m * m


def _mm_body(x_ref, w_ref, o_ref):
    o_ref[...] = jnp.dot(x_ref[...], w_ref[...], preferred_element_type=jnp.float32)


def _matmul(x, w):
    npad, cp = x.shape[0], w.shape[1]
    nb = npad // 512
    return pl.pallas_call(
        _mm_body,
        grid=(nb,),
        in_specs=[pl.BlockSpec((512, _H), lambda i: (i, 0)),
                  pl.BlockSpec((_H, cp), lambda i: (0, 0))],
        out_specs=pl.BlockSpec((512, cp), lambda i: (i, 0)),
        out_shape=jax.ShapeDtypeStruct((npad, cp), jnp.float32),
    )(x, w)


def _pool_body(ids_ref, x_ref, o_ref):
    @pl.when(pl.program_id(0) == 0)
    def _():
        o_ref[...] = jnp.zeros_like(o_ref)
    ids = ids_ref[0, 0, :]
    mask = (ids[None, :] == lax.broadcasted_iota(jnp.int32, (_B, 512), 0)).astype(jnp.float32)
    o_ref[...] += jnp.dot(mask, x_ref[...], preferred_element_type=jnp.float32)


def _pool_sums(x, ids):
    n = x.shape[0]
    npad = _round_up(n, 512)
    xp = jnp.concatenate([x, jnp.ones((n, 1), jnp.float32)], axis=1)
    xp = jnp.pad(xp, ((0, npad - n), (0, 256 - _H - 1)))
    idp = jnp.pad(ids.astype(jnp.int32), (0, npad - n), constant_values=_B)
    idp = idp.reshape(npad // 512, 1, 512)
    nb = npad // 512
    return pl.pallas_call(
        _pool_body,
        grid=(nb,),
        in_specs=[pl.BlockSpec((1, 1, 512), lambda i: (i, 0, 0)),
                  pl.BlockSpec((512, 256), lambda i: (i, 0))],
        out_specs=pl.BlockSpec((_B, 256), lambda i: (0, 0)),
        out_shape=jax.ShapeDtypeStruct((_B, 256), jnp.float32),
    )(idp, xp)


def _head_body(sv_ref, sc_ref, w_ref, b_ref, o_ref):
    sv = sv_ref[...]
    sc = sc_ref[...]
    cv = jnp.maximum(sv[:, _H:_H + 1], 1.0)
    cc = jnp.maximum(sc[:, _H:_H + 1], 1.0)
    pooled = jnp.concatenate([sv[:, :_H] / cv, sc[:, :_H] / cc], axis=1)
    o_ref[...] = jnp.dot(pooled, w_ref[...], preferred_element_type=jnp.float32) + b_ref[...]


def _head(sv, sc, lin_w, lin_b):
    wp = jnp.pad(lin_w, ((0, 0), (0, _H - lin_w.shape[1])))
    bp = jnp.pad(lin_b, (0, _H - lin_b.shape[0]))[None, :]
    out = pl.pallas_call(
        _head_body,
        out_shape=jax.ShapeDtypeStruct((_B, _H), jnp.float32),
    )(sv, sc, wp, bp)
    return out[:, :lin_w.shape[1]]


def _make_sc_kernel(k_rels, w_workers, rows, ns_cat, op_cols):
    mesh = plsc.VectorSubcoreMesh(core_axis_name="c", subcore_axis_name="s")
    nrounds = w_workers // 32
    ndp = w_workers * rows

    @functools.partial(
        pl.kernel,
        mesh=mesh,
        compiler_params=pltpu.CompilerParams(needs_layout_passes=False),
        out_type=jax.ShapeDtypeStruct((ndp * _H,), jnp.float32),
        scratch_types=[
            pltpu.VMEM((rows * _H,), jnp.float32),
            pltpu.VMEM((2, _C, _H), jnp.float32),
            pltpu.VMEM((2, _C), jnp.int32),
            pltpu.VMEM((2, _C), jnp.int32),
            pltpu.VMEM((2, _C), jnp.float32),
            pltpu.VMEM((_C,), jnp.float32),
            pltpu.VMEM((_C,), jnp.int32),
            pltpu.VMEM((rows + 128,), jnp.float32),
            pltpu.VMEM((rows,), jnp.float32),
            pltpu.VMEM((len(k_rels), op_cols), jnp.int32),
            pltpu.VMEM((16,), jnp.float32),
            pltpu.SemaphoreType.DMA,
            pltpu.SemaphoreType.DMA,
        ],
    )
    def sc_kernel(hs_hbm, ssrc_hbm, sdst_hbm, src_hbm, dst_hbm, offs_hbm, m_hbm,
                  out_hbm,
                  out_v, rows_v, srcidx_v, dstidx_v, ssrcg_v, ee_v, dl_v,
                  sdst_v, den_v, offs_s, m_s, sem, sem_i):
        wid = lax.axis_index("s") * 2 + lax.axis_index("c")
        pltpu.sync_copy(offs_hbm, offs_s)
        pltpu.sync_copy(m_hbm, m_s)
        lanes = lax.iota(jnp.int32, 16)
        lane0 = lanes == 0
        zero16 = jnp.zeros((16,), jnp.float32)
        mvec = m_s[...]

        def edge_scalars(kk, p, base, d0, doff, e0, e1, j):
            gidx = base + j * 16 + lanes
            valid = (gidx >= e0) & (gidx < e1)
            dst16 = dstidx_v[p, pl.ds(j * 16, 16)]
            dl16 = jnp.clip(dst16 - d0, 0, rows - 1)
            ssrc16 = ssrcg_v[p, pl.ds(j * 16, 16)]
            sdst16 = plsc.load_gather(sdst_v, [dl16 + doff])
            s = ssrc16 + sdst16
            e = jnp.where(s > 0, s, 0.2 * s)
            ee = jnp.exp(e - mvec[kk])
            ee = jnp.where(valid, ee, 0.0)
            return dl16, ee

        def issue_idx(kk, base, p):
            pltpu.async_copy(src_hbm.at[pl.ds(kk * _EPAD + base, _C)],
                             srcidx_v.at[p], sem_i)
            pltpu.async_copy(dst_hbm.at[pl.ds(kk * _EPAD + base, _C)],
                             dstidx_v.at[p], sem_i)

        def wait_idx(kk, base, p):
            pltpu.make_async_copy(src_hbm.at[pl.ds(kk * _EPAD + base, _C)],
                                  srcidx_v.at[p], sem_i).wait()
            pltpu.make_async_copy(dst_hbm.at[pl.ds(kk * _EPAD + base, _C)],
                                  dstidx_v.at[p], sem_i).wait()

        def issue_gather(p, with_rows):
            pltpu.async_copy(ssrc_hbm.at[srcidx_v.at[p]], ssrcg_v.at[p], sem)
            if with_rows:
                pltpu.async_copy(hs_hbm.at[srcidx_v.at[p]], rows_v.at[p], sem)

        def wait_gather(p, with_rows):
            pltpu.make_async_copy(ssrc_hbm.at[srcidx_v.at[p]], ssrcg_v.at[p], sem).wait()
            if with_rows:
                pltpu.make_async_copy(hs_hbm.at[srcidx_v.at[p]], rows_v.at[p], sem).wait()

        def run_pass(kk, a0, nch, with_rows, compute):
            cbase = lambda c: pl.multiple_of(a0 + c * _C, 128)
            issue_idx(kk, cbase(0), 0)
            wait_idx(kk, cbase(0), 0)
            issue_gather(0, with_rows)
            issue_idx(kk, cbase(1), 1)

            def step(c, _c):
                p = c & 1
                q = 1 - p
                wait_gather(p, with_rows)
                wait_idx(kk, cbase(c + 1), q)
                issue_gather(q, with_rows)
                compute(c, p)
                issue_idx(kk, cbase(c + 2), p)
                return _c
            lax.fori_loop(0, nch, step, 0)
            pe = nch & 1
            wait_gather(pe, with_rows)
            wait_idx(kk, cbase(nch + 1), 1 - pe)

        def per_worker(r, _):
            w = r * 32 + wid
            d0 = w * rows
            d0a = pl.multiple_of((d0 >> 7) << 7, 128)
            doff = d0 - d0a
            def zrow(i, _c):
                out_v[pl.ds(pl.multiple_of(i * 16, 16), 16)] = zero16
                return _c
            lax.fori_loop(0, rows * _H // 16, zrow, 0)

            for kk in range(len(k_rels)):
                evec = offs_s[kk, pl.ds(w, 16)]
                e0 = evec[0]
                e1 = evec[1]
                a0 = (e0 >> 7) << 7
                nch = (e1 - a0 + _C - 1) // _C
                pltpu.sync_copy(sdst_hbm.at[pl.ds(pl.multiple_of(kk * ndp + d0a, 128), rows + 128)], sdst_v)
                def zden(i, _c):
                    den_v[pl.ds(pl.multiple_of(i * 16, 16), 16)] = zero16
                    return _c
                lax.fori_loop(0, rows // 16, zden, 0)

                def compute1(c, p):
                    base = pl.multiple_of(a0 + c * _C, 128)
                    for j in range(_C // 16):
                        dl16, ee = edge_scalars(kk, p, base, d0, doff, e0, e1, j)
                        ee_v[pl.ds(j * 16, 16)] = ee
                        dl_v[pl.ds(j * 16, 16)] = dl16
                    def acc(i, _a):
                        bidx = jnp.full((16,), i, jnp.int32)
                        dlb = plsc.load_gather(dl_v, [bidx])
                        eeb = plsc.load_gather(ee_v, [bidx])
                        plsc.addupdate_scatter(den_v, [dlb], eeb, mask=lane0)
                        return _a
                    lax.fori_loop(0, _C, acc, 0)
                run_pass(kk, a0, nch, False, compute1)

                def compute2(c, p):
                    base = pl.multiple_of(a0 + c * _C, 128)
                    for j in range(_C // 16):
                        dl16, ee = edge_scalars(kk, p, base, d0, doff, e0, e1, j)
                        den16 = plsc.load_gather(den_v, [dl16])
                        alpha = ee / (den16 + 1e-16)
                        ee_v[pl.ds(j * 16, 16)] = alpha
                        dl_v[pl.ds(j * 16, 16)] = dl16
                    def acc(i, _a):
                        bidx = jnp.full((16,), i, jnp.int32)
                        dlb = plsc.load_gather(dl_v, [bidx])
                        ab = plsc.load_gather(ee_v, [bidx])
                        obase = dlb * _H + lanes
                        for j in range(_H // 16):
                            vals = rows_v[p, i, pl.ds(j * 16, 16)] * ab
                            plsc.addupdate_scatter(out_v, [obase + j * 16], vals)
                        return _a
                    lax.fori_loop(0, _C, acc, 0)
                run_pass(kk, a0, nch, True, compute2)

            pltpu.sync_copy(out_v, out_hbm.at[pl.ds(pl.multiple_of(d0 * _H, 128), rows * _H)])
            return _

        lax.fori_loop(0, nrounds, per_worker, 0)

    return sc_kernel


def kernel(x_variable, x_value, x_operator, x_constraint, edge_index_0, edge_index_1,
           edge_index_2, edge_index_3, edge_index_4, edge_index_5, edge_index_6,
           edge_index_7, edge_index_8, batch_variable, batch_constraint,
           Wsrc, Wdst, att_src, att_dst, bias, linW, linb):
    xs = {'variable': x_variable, 'value': x_value, 'operator': x_operator,
          'constraint': x_constraint}
    edges = [edge_index_0, edge_index_1, edge_index_2, edge_index_3, edge_index_4,
             edge_index_5, edge_index_6, edge_index_7, edge_index_8]

    src_rels = {t: [r for r, (s, d) in enumerate(_RELS) if s == t] for t in _TYPES}
    dst_rels = {t: [r for r, (s, d) in enumerate(_RELS) if d == t] for t in _TYPES}

    srt = {}
    for dtp, rels in _GROUPS.items():
        w_workers, rows = _WCFG[dtp]
        op_cols = w_workers + 16
        base = 0
        src_list, dst_list, off_list, bases = [], [], [], []
        for r in rels:
            src, dst = edges[r][0].astype(jnp.int32), edges[r][1].astype(jnp.int32)
            order = jnp.argsort(dst)
            dst_s = dst[order]
            src_s = src[order] + base
            offs = jnp.searchsorted(dst_s, jnp.arange(w_workers + 1, dtype=jnp.int32) * rows
                                    ).astype(jnp.int32)
            offs = jnp.pad(offs, (0, op_cols - (w_workers + 1)), constant_values=_E)
            src_list.append(jnp.pad(src_s, (0, _EPAD - _E)))
            dst_list.append(jnp.pad(dst_s, (0, _EPAD - _E), constant_values=w_workers * rows))
            off_list.append(offs)
            bases.append(base)
            base += _NT[_RELS[r][0]]
        srt[dtp] = (jnp.concatenate(src_list), jnp.concatenate(dst_list),
                    jnp.stack(off_list), bases)

    sc_kernels = {dtp: _make_sc_kernel(rels, _WCFG[dtp][0], _WCFG[dtp][1],
                                       None, _WCFG[dtp][0] + 16)
                  for dtp, rels in _GROUPS.items()}

    x = {t: v for t, v in xs.items()}
    for l in range(_L):
        hs = {}
        ssrc = {}
        sdst = {}
        for t in _TYPES:
            n = _NT[t]
            npad = _round_up(n, 512)
            cols = []
            for r in src_rels[t]:
                cols.append(Wsrc[l, r])
            for r in src_rels[t]:
                cols.append((Wsrc[l, r] @ att_src[l, r])[:, None])
            for r in dst_rels[t]:
                cols.append((Wdst[l, r] @ att_dst[l, r])[:, None])
            wcat = jnp.concatenate(cols, axis=1)
            cp = _round_up(wcat.shape[1], _H)
            wcat = jnp.pad(wcat, ((0, 0), (0, cp - wcat.shape[1])))
            xp = jnp.pad(x[t], ((0, npad - x[t].shape[0]), (0, 0)))
            y = _matmul(xp, wcat)
            ks = len(src_rels[t])
            for i, r in enumerate(src_rels[t]):
                hs[r] = y[:n, i * _H:(i + 1) * _H]
                ssrc[r] = y[:n, ks * _H + i]
            for j, r in enumerate(dst_rels[t]):
                sdst[r] = y[:n, ks * _H + ks + j]

        acc = {}
        for dtp, rels in _GROUPS.items():
            w_workers, rows = _WCFG[dtp]
            ndp = w_workers * rows
            src_st, dst_st, off_st, bases = srt[dtp]
            hs_cat = jnp.concatenate([hs[r] for r in rels], axis=0)
            ssrc_cat = jnp.concatenate([ssrc[r] for r in rels], axis=0)
            sdst_all = jnp.concatenate([jnp.pad(sdst[r], (0, ndp - _NT[dtp])) for r in rels]
                                       + [jnp.zeros((128,), jnp.float32)])
            m_k = []
            for r in rels:
                s = jnp.max(ssrc[r]) + jnp.max(sdst[r])
                m_k.append(jnp.where(s > 0, s, 0.2 * s))
            m_k = jnp.stack(m_k)
            m_k = jnp.pad(m_k, (0, 16 - len(rels)))
            out = sc_kernels[dtp](hs_cat, ssrc_cat, sdst_all, src_st, dst_st,
                                  off_st, m_k)
            out = out.reshape(ndp, _H)
            bsum = sum(bias[l, r] for r in rels)
            acc[dtp] = out[:_NT[dtp]] + bsum[None, :]
        x = {t: jax.nn.relu(v) for t, v in acc.items()}

    sv = _pool_sums(x['variable'], batch_variable)
    sc_ = _pool_sums(x['constraint'], batch_constraint)
    return _head(sv, sc_, linW, linb)
